# baseline (speedup 1.0000x reference)
import functools

import jax
import jax.numpy as jnp
from jax.experimental import pallas as pl
from jax.experimental.pallas import tpu as pltpu

_BI = 512
_BJ = 512


def _sage_layer_kernel(g_ref, hsrc_ref, hdst_ref, ws_ref, wn_ref, b_ref,
                       out_ref, acc_ref, deg_ref, *, nsteps, relu):
    i = pl.program_id(1)

    @pl.when(i == 0)
    def _init():
        acc_ref[...] = jnp.zeros_like(acc_ref)
        deg_ref[...] = jnp.zeros_like(deg_ref)

    gb = g_ref[...].astype(jnp.bfloat16)
    hb = hsrc_ref[...].astype(jnp.bfloat16)
    acc_ref[...] += jax.lax.dot_general(
        gb, hb, (((0,), (0,)), ((), ())),
        preferred_element_type=jnp.float32)
    ones = jnp.ones((gb.shape[0], 1), dtype=jnp.bfloat16)
    deg_ref[...] += jax.lax.dot_general(
        gb, ones, (((0,), (0,)), ((), ())),
        preferred_element_type=jnp.float32)

    @pl.when(i == nsteps - 1)
    def _finalize():
        inv = 1.0 / jnp.maximum(deg_ref[...], 1.0)
        neigh = acc_ref[...] * inv
        hd = hdst_ref[...].astype(jnp.bfloat16)
        ws = ws_ref[...].astype(jnp.bfloat16)
        wn = wn_ref[...].astype(jnp.bfloat16)
        out = (jax.lax.dot_general(hd, ws, (((1,), (0,)), ((), ())),
                                   preferred_element_type=jnp.float32)
               + jax.lax.dot_general(neigh.astype(jnp.bfloat16), wn,
                                     (((1,), (0,)), ((), ())),
                                     preferred_element_type=jnp.float32)
               + b_ref[...])
        if relu:
            out = jnp.maximum(out, 0.0)
        out_ref[...] = out


def _sage_layer(graph, h, ws, wn, b, relu):
    n, d_in = h.shape
    d_out = ws.shape[1]
    ni = n // _BI
    nj = n // _BJ
    kern = functools.partial(_sage_layer_kernel, nsteps=ni, relu=relu)
    return pl.pallas_call(
        kern,
        grid=(nj, ni),
        in_specs=[
            pl.BlockSpec((_BI, _BJ), lambda j, i: (i, j)),
            pl.BlockSpec((_BI, d_in), lambda j, i: (i, 0)),
            pl.BlockSpec((_BJ, d_in), lambda j, i: (j, 0)),
            pl.BlockSpec((d_in, d_out), lambda j, i: (0, 0)),
            pl.BlockSpec((d_in, d_out), lambda j, i: (0, 0)),
            pl.BlockSpec((1, d_out), lambda j, i: (0, 0)),
        ],
        out_specs=pl.BlockSpec((_BJ, d_out), lambda j, i: (j, 0)),
        out_shape=jax.ShapeDtypeStruct((n, d_out), jnp.float32),
        scratch_shapes=[
            pltpu.VMEM((_BJ, d_in), jnp.float32),
            pltpu.VMEM((_BJ, 1), jnp.float32),
        ],
        compiler_params=pltpu.CompilerParams(
            dimension_semantics=("parallel", "arbitrary")),
    )(graph, h, h, ws, wn, b.reshape(1, -1))


def kernel(inputs, graph, W_self1, W_neigh1, b1, W_self2, W_neigh2, b2):
    h1 = _sage_layer(graph, inputs, W_self1, W_neigh1, b1, relu=True)
    return _sage_layer(graph, h1, W_self2, W_neigh2, b2, relu=False)


# speedup vs baseline: 1.4621x; 1.4621x over previous
import functools

import jax
import jax.numpy as jnp
from jax.experimental import pallas as pl
from jax.experimental.pallas import tpu as pltpu

_BI = 512
_BJ = 512


def _fused_kernel(g_ref, x_ref, ws1_ref, wn1_ref, b1_ref, ws2_ref, wn2_ref,
                  b2_ref, out_ref, gbf_ref, h1_ref, acct_ref, deg_ref,
                  inv_ref, *, ni):
    l = pl.program_id(0)
    j = pl.program_id(1)
    i = pl.program_id(2)

    @pl.when(i == 0)
    def _init():
        acct_ref[...] = jnp.zeros_like(acct_ref)
        deg_ref[...] = jnp.zeros_like(deg_ref)

    @pl.when(l == 0)
    def _layer0_step():
        gb = g_ref[...].astype(jnp.bfloat16)
        gbf_ref[pl.ds(i * _BI, _BI), pl.ds(j * _BJ, _BJ)] = gb
        hb = x_ref[pl.ds(i * _BI, _BI), :].astype(jnp.bfloat16)
        acct_ref[...] += jax.lax.dot_general(
            hb, gb, (((0,), (0,)), ((), ())),
            preferred_element_type=jnp.float32)
        deg_ref[...] += jnp.sum(g_ref[...], axis=0, keepdims=True)

        @pl.when(i == ni - 1)
        def _fin0():
            inv = 1.0 / jnp.maximum(deg_ref[...], 1.0)
            inv_ref[:, pl.ds(j * _BJ, _BJ)] = inv
            neight = (acct_ref[...] * inv).astype(jnp.bfloat16)
            hd = x_ref[pl.ds(j * _BJ, _BJ), :].astype(jnp.bfloat16)
            h1 = (jax.lax.dot_general(
                      hd, ws1_ref[...], (((1,), (0,)), ((), ())),
                      preferred_element_type=jnp.float32)
                  + jax.lax.dot_general(
                      neight, wn1_ref[...], (((0,), (0,)), ((), ())),
                      preferred_element_type=jnp.float32)
                  + b1_ref[...])
            h1_ref[pl.ds(j * _BJ, _BJ), :] = jnp.maximum(h1, 0.0)

    @pl.when(l == 1)
    def _layer1_step():
        gb = gbf_ref[pl.ds(i * _BI, _BI), pl.ds(j * _BJ, _BJ)]
        hb = h1_ref[pl.ds(i * _BI, _BI), :].astype(jnp.bfloat16)
        acct_ref[...] += jax.lax.dot_general(
            hb, gb, (((0,), (0,)), ((), ())),
            preferred_element_type=jnp.float32)

        @pl.when(i == ni - 1)
        def _fin1():
            inv = inv_ref[:, pl.ds(j * _BJ, _BJ)]
            neight = (acct_ref[...] * inv).astype(jnp.bfloat16)
            hd = h1_ref[pl.ds(j * _BJ, _BJ), :].astype(jnp.bfloat16)
            out = (jax.lax.dot_general(
                       hd, ws2_ref[...], (((1,), (0,)), ((), ())),
                       preferred_element_type=jnp.float32)
                   + jax.lax.dot_general(
                       neight, wn2_ref[...], (((0,), (0,)), ((), ())),
                       preferred_element_type=jnp.float32)
                   + b2_ref[...])
            out_ref[...] = out


def kernel(inputs, graph, W_self1, W_neigh1, b1, W_self2, W_neigh2, b2):
    n, d_in = inputs.shape
    d_hid = W_self1.shape[1]
    d_out = W_self2.shape[1]
    ni = n // _BI
    nj = n // _BJ
    kern = functools.partial(_fused_kernel, ni=ni)
    ws1b = W_self1.astype(jnp.bfloat16)
    wn1b = W_neigh1.astype(jnp.bfloat16)
    ws2b = W_self2.astype(jnp.bfloat16)
    wn2b = W_neigh2.astype(jnp.bfloat16)
    return pl.pallas_call(
        kern,
        grid=(2, nj, ni),
        in_specs=[
            pl.BlockSpec((_BI, _BJ),
                         lambda l, j, i: (jnp.where(l == 0, i, 0),
                                          jnp.where(l == 0, j, 0))),
            pl.BlockSpec((n, d_in), lambda l, j, i: (0, 0)),
            pl.BlockSpec((d_in, d_hid), lambda l, j, i: (0, 0)),
            pl.BlockSpec((d_in, d_hid), lambda l, j, i: (0, 0)),
            pl.BlockSpec((1, d_hid), lambda l, j, i: (0, 0)),
            pl.BlockSpec((d_hid, d_out), lambda l, j, i: (0, 0)),
            pl.BlockSpec((d_hid, d_out), lambda l, j, i: (0, 0)),
            pl.BlockSpec((1, d_out), lambda l, j, i: (0, 0)),
        ],
        out_specs=pl.BlockSpec((_BJ, d_out),
                               lambda l, j, i: (jnp.where(l == 0, 0, j), 0)),
        out_shape=jax.ShapeDtypeStruct((n, d_out), jnp.float32),
        scratch_shapes=[
            pltpu.VMEM((n, n), jnp.bfloat16),
            pltpu.VMEM((n, d_hid), jnp.float32),
            pltpu.VMEM((d_in, _BJ), jnp.float32),
            pltpu.VMEM((1, _BJ), jnp.float32),
            pltpu.VMEM((1, n), jnp.float32),
        ],
        compiler_params=pltpu.CompilerParams(
            dimension_semantics=("arbitrary", "arbitrary", "arbitrary")),
    )(graph, inputs, ws1b, wn1b, b1.reshape(1, -1), ws2b, wn2b,
      b2.reshape(1, -1))


# speedup vs baseline: 3.1534x; 2.1568x over previous
import functools

import jax
import jax.numpy as jnp
from jax.experimental import pallas as pl
from jax.experimental.pallas import tpu as pltpu

_BJ = 512


def _fused_kernel(g_ref, x_ref, ws1_ref, wn1_ref, b1_ref, ws2_ref, wn2_ref,
                  b2_ref, out_ref, gbf_ref, xaug_ref, h1_ref, inv_ref):
    l = pl.program_id(0)
    j = pl.program_id(1)
    n, d_in = x_ref.shape

    @pl.when(l == 0)
    def _layer0():
        @pl.when(j == 0)
        def _stage_x():
            xaug_ref[:, :d_in] = x_ref[...].astype(jnp.bfloat16)
            xaug_ref[:, d_in:] = jnp.ones((n, 1), jnp.bfloat16)

        gbf_ref[:, pl.ds(j * _BJ, _BJ)] = g_ref[...].astype(jnp.bfloat16)
        gb = gbf_ref[:, pl.ds(j * _BJ, _BJ)]
        acct = jax.lax.dot_general(
            xaug_ref[...], gb, (((0,), (0,)), ((), ())),
            preferred_element_type=jnp.float32)
        inv = 1.0 / jnp.maximum(acct[d_in:, :], 1.0)
        inv_ref[:, pl.ds(j * _BJ, _BJ)] = inv
        neight = (acct[:d_in, :] * inv).astype(jnp.bfloat16)
        hd = xaug_ref[pl.ds(j * _BJ, _BJ), :d_in]
        h1 = (jax.lax.dot_general(
                  hd, ws1_ref[...], (((1,), (0,)), ((), ())),
                  preferred_element_type=jnp.float32)
              + jax.lax.dot_general(
                  neight, wn1_ref[...], (((0,), (0,)), ((), ())),
                  preferred_element_type=jnp.float32)
              + b1_ref[...])
        h1_ref[pl.ds(j * _BJ, _BJ), :] = jnp.maximum(h1, 0.0).astype(
            jnp.bfloat16)

    @pl.when(l == 1)
    def _layer1():
        gb = gbf_ref[:, pl.ds(j * _BJ, _BJ)]
        acct = jax.lax.dot_general(
            h1_ref[...], gb, (((0,), (0,)), ((), ())),
            preferred_element_type=jnp.float32)
        inv = inv_ref[:, pl.ds(j * _BJ, _BJ)]
        neight = (acct * inv).astype(jnp.bfloat16)
        hd = h1_ref[pl.ds(j * _BJ, _BJ), :]
        out = (jax.lax.dot_general(
                   hd, ws2_ref[...], (((1,), (0,)), ((), ())),
                   preferred_element_type=jnp.float32)
               + jax.lax.dot_general(
                   neight, wn2_ref[...], (((0,), (0,)), ((), ())),
                   preferred_element_type=jnp.float32)
               + b2_ref[...])
        out_ref[...] = out


def kernel(inputs, graph, W_self1, W_neigh1, b1, W_self2, W_neigh2, b2):
    n, d_in = inputs.shape
    d_hid = W_self1.shape[1]
    d_out = W_self2.shape[1]
    nj = n // _BJ
    ws1b = W_self1.astype(jnp.bfloat16)
    wn1b = W_neigh1.astype(jnp.bfloat16)
    ws2b = W_self2.astype(jnp.bfloat16)
    wn2b = W_neigh2.astype(jnp.bfloat16)
    return pl.pallas_call(
        _fused_kernel,
        grid=(2, nj),
        in_specs=[
            pl.BlockSpec((n, _BJ), lambda l, j: (0, jnp.where(l == 0, j, 0))),
            pl.BlockSpec((n, d_in), lambda l, j: (0, 0)),
            pl.BlockSpec((d_in, d_hid), lambda l, j: (0, 0)),
            pl.BlockSpec((d_in, d_hid), lambda l, j: (0, 0)),
            pl.BlockSpec((1, d_hid), lambda l, j: (0, 0)),
            pl.BlockSpec((d_hid, d_out), lambda l, j: (0, 0)),
            pl.BlockSpec((d_hid, d_out), lambda l, j: (0, 0)),
            pl.BlockSpec((1, d_out), lambda l, j: (0, 0)),
        ],
        out_specs=pl.BlockSpec((_BJ, d_out),
                               lambda l, j: (jnp.where(l == 0, 0, j), 0)),
        out_shape=jax.ShapeDtypeStruct((n, d_out), jnp.float32),
        scratch_shapes=[
            pltpu.VMEM((n, n), jnp.bfloat16),
            pltpu.VMEM((n, d_in + 1), jnp.bfloat16),
            pltpu.VMEM((n, d_hid), jnp.bfloat16),
            pltpu.VMEM((1, n), jnp.float32),
        ],
        compiler_params=pltpu.CompilerParams(
            dimension_semantics=("arbitrary", "arbitrary")),
    )(graph, inputs, ws1b, wn1b, b1.reshape(1, -1), ws2b, wn2b,
      b2.reshape(1, -1))
